# baseline (speedup 1.0000x reference)
import functools

import jax
import jax.numpy as jnp
from jax import lax
from jax.experimental import pallas as pl
from jax.experimental.pallas import tpu as pltpu
from jax.experimental.pallas import tpu_sc as plsc

N_ROWS = 1_000_000
COLS = 7
LANES = 16
NC, NS = 2, 16
NW = NC * NS
ROWS_PER_CHUNK = 800
FLAT_PER_CHUNK = ROWS_PER_CHUNK * COLS
NUM_CHUNKS = N_ROWS // ROWS_PER_CHUNK
GROUPS = ROWS_PER_CHUNK // LANES
THRESH = 0.55


def _sc_body(pred_hbm, target_hbm, iou_hbm, ws_out, cnt_out,
             pbuf, tbuf, ibuf, mbuf, obuf):
  wid = lax.axis_index("s") * NC + lax.axis_index("c")

  iota = lax.iota(jnp.int32, LANES)
  row_idx = [(iota + 16 * j) // COLS for j in range(COLS)]

  base_chunks = NUM_CHUNKS // NW
  extra = NUM_CHUNKS - base_chunks * NW
  n_my = base_chunks + jnp.where(wid < extra, 1, 0)

  zero = jnp.zeros((LANES,), jnp.float32)

  def chunk_body(i, carry):
    accr, accm, cacc = carry
    chunk = wid + i * NW
    pltpu.sync_copy(pred_hbm.at[pl.ds(chunk * FLAT_PER_CHUNK, FLAT_PER_CHUNK)],
                    pbuf)
    pltpu.sync_copy(target_hbm.at[pl.ds(chunk * FLAT_PER_CHUNK, FLAT_PER_CHUNK)],
                    tbuf)
    pltpu.sync_copy(iou_hbm.at[pl.ds(chunk * ROWS_PER_CHUNK, ROWS_PER_CHUNK)],
                    ibuf)

    def group_body(g, c2):
      accr2, accm2, cacc2 = c2
      m16 = jnp.where(ibuf[pl.ds(g * LANES, LANES)] >= THRESH, 1.0, 0.0)
      cacc2 = cacc2 + m16
      mbuf[pl.ds(0, LANES)] = m16
      accr3, accm3 = [], []
      for j in range(COLS):
        off = g * (LANES * COLS) + j * LANES
        d = pbuf[pl.ds(off, LANES)] - tbuf[pl.ds(off, LANES)]
        dm = d * plsc.load_gather(mbuf, [row_idx[j]])
        ax = jnp.abs(dm)
        mn = jnp.minimum(ax, 1.0)
        accr3.append(accr2[j] + (ax - mn))
        accm3.append(accm2[j] + mn * mn)
      return tuple(accr3), tuple(accm3), cacc2

    return lax.fori_loop(0, GROUPS, group_body, (accr, accm, cacc))

  accr, accm, cacc = lax.fori_loop(
      0, n_my, chunk_body,
      ((zero,) * COLS, (zero,) * COLS, zero))

  ws = zero
  for j in range(COLS):
    col = (iota + 16 * j) % COLS
    w = jnp.where(col < 3, jnp.float32(1.0 / 3.0), jnp.float32(1.0))
    ws = ws + w * (0.5 * accm[j] + accr[j])

  obuf[pl.ds(0, LANES)] = ws
  obuf[pl.ds(LANES, LANES)] = cacc
  pltpu.sync_copy(obuf.at[pl.ds(0, LANES)],
                  ws_out.at[pl.ds(wid * LANES, LANES)])
  pltpu.sync_copy(obuf.at[pl.ds(LANES, LANES)],
                  cnt_out.at[pl.ds(wid * LANES, LANES)])


@functools.partial(
    pl.kernel,
    out_type=(jax.ShapeDtypeStruct((NW * LANES,), jnp.float32),
              jax.ShapeDtypeStruct((NW * LANES,), jnp.float32)),
    mesh=plsc.VectorSubcoreMesh(core_axis_name="c", subcore_axis_name="s",
                                num_cores=NC, num_subcores=NS),
    scratch_types=(
        pltpu.VMEM((FLAT_PER_CHUNK,), jnp.float32),
        pltpu.VMEM((FLAT_PER_CHUNK,), jnp.float32),
        pltpu.VMEM((ROWS_PER_CHUNK,), jnp.float32),
        pltpu.VMEM((LANES,), jnp.float32),
        pltpu.VMEM((2 * LANES,), jnp.float32),
    ),
    compiler_params=pltpu.CompilerParams(needs_layout_passes=False),
)
def _sc_loss(pred_hbm, target_hbm, iou_hbm, ws_out, cnt_out,
             pbuf, tbuf, ibuf, mbuf, obuf):
  _sc_body(pred_hbm, target_hbm, iou_hbm, ws_out, cnt_out,
           pbuf, tbuf, ibuf, mbuf, obuf)


def kernel(pred, target, iou):
  ws, cnt = _sc_loss(pred.reshape(-1), target.reshape(-1), iou)
  return jnp.sum(ws) / jnp.sum(cnt)


# speedup vs baseline: 14.4100x; 14.4100x over previous
import functools

import jax
import jax.numpy as jnp
from jax import lax
from jax.experimental import pallas as pl
from jax.experimental.pallas import tpu as pltpu
from jax.experimental.pallas import tpu_sc as plsc

N_ROWS = 1_000_000
COLS = 7
LANES = 16
NC, NS = 2, 16
NW = NC * NS
R = 512
N_MAIN = (N_ROWS // 128) // (R // 128) * R
C = N_MAIN // R
N_TAIL = N_ROWS - N_MAIN
GROUPS = R // LANES
TGROUPS = N_TAIL // LANES
NMAX = -(-C // NW)
NMAX += NMAX % 2
THRESH = 0.55
INF = float("inf")


def _sc_body(pred_hbm, target_hbm, iou_hbm, ptail_hbm, ttail_hbm,
             ws_out, cnt_out,
             pbuf, tbuf, ibuf, ptb, ttb, itb, mbuf, obuf,
             sem0, sem1, sem2):
  wid = lax.axis_index("s") * NC + lax.axis_index("c")
  sems = (sem0, sem1)

  def chunk_of(i):
    return jnp.minimum(wid + NW * i, C - 1)

  def copies(slot, chunk):
    r0 = chunk * R
    sem = sems[slot]
    return [
        pltpu.make_async_copy(pred_hbm.at[:, pl.ds(r0, R)],
                              pbuf.at[slot], sem),
        pltpu.make_async_copy(target_hbm.at[:, pl.ds(r0, R)],
                              tbuf.at[slot], sem),
        pltpu.make_async_copy(iou_hbm.at[pl.ds(r0, R)],
                              ibuf.at[slot], sem),
    ]

  def issue(slot, chunk):
    for cp in copies(slot, chunk):
      cp.start()

  def drain(slot):
    for cp in copies(slot, 0):
      cp.wait()

  tail_copies = [
      pltpu.make_async_copy(ptail_hbm, ptb, sem2),
      pltpu.make_async_copy(ttail_hbm, ttb, sem2),
      pltpu.make_async_copy(iou_hbm.at[pl.ds(N_MAIN, N_TAIL)], itb, sem2),
  ]
  for cp in tail_copies:
    cp.start()

  issue(0, chunk_of(0))

  zero = jnp.zeros((LANES,), jnp.float32)

  def pair_body(p, carry):
    for s in (0, 1):
      i = 2 * p + s
      nxt = i + 1

      @pl.when(nxt < NMAX)
      def _():
        issue(s ^ 1, chunk_of(nxt))

      drain(s)
      t_eff = jnp.where(wid + NW * i <= C - 1, THRESH, INF)
      accs, cacc = carry

      def group_body(g, c2):
        accs2, cacc2 = c2
        m16 = jnp.where(ibuf[s, pl.ds(g * LANES, LANES)] >= t_eff, 1.0, 0.0)
        cacc2 = cacc2 + m16
        accs3 = []
        for c in range(COLS):
          d = (pbuf[s, c, pl.ds(g * LANES, LANES)]
               - tbuf[s, c, pl.ds(g * LANES, LANES)])
          dm = d * m16
          ax = jnp.abs(dm)
          mn = jnp.minimum(ax, 1.0)
          accs3.append(accs2[c] + mn * (ax + ax - mn))
        return tuple(accs3), cacc2

      carry = lax.fori_loop(0, GROUPS, group_body, (accs, cacc))
    return carry

  accs, cacc = lax.fori_loop(0, NPAIRS, pair_body, ((zero,) * COLS, zero))

  ws = zero
  for c in range(COLS):
    ws = ws + jnp.float32(1.0 / 6.0 if c < 3 else 0.5) * accs[c]

  for cp in tail_copies:
    cp.wait()
  iota = lax.iota(jnp.int32, LANES)
  t_tail = jnp.where(wid == NW - 1, THRESH, INF)
  for g in range(TGROUPS):
    m16 = jnp.where(itb[pl.ds(g * LANES, LANES)] >= t_tail, 1.0, 0.0)
    cacc = cacc + m16
    mbuf[pl.ds(0, LANES)] = m16
    for j in range(COLS):
      k = iota + (g * COLS + j) * LANES
      row_in_g = (k // COLS) - g * LANES
      col = k % COLS
      wj = jnp.where(col < 3, jnp.float32(1.0 / 6.0), jnp.float32(0.5))
      off = (g * COLS + j) * LANES
      d = ptb[pl.ds(off, LANES)] - ttb[pl.ds(off, LANES)]
      dm = d * plsc.load_gather(mbuf, [row_in_g])
      ax = jnp.abs(dm)
      mn = jnp.minimum(ax, 1.0)
      ws = ws + wj * (mn * (ax + ax - mn))

  obuf[pl.ds(0, LANES)] = ws
  obuf[pl.ds(LANES, LANES)] = cacc
  pltpu.sync_copy(obuf.at[pl.ds(0, LANES)],
                  ws_out.at[pl.ds(wid * LANES, LANES)])
  pltpu.sync_copy(obuf.at[pl.ds(LANES, LANES)],
                  cnt_out.at[pl.ds(wid * LANES, LANES)])


NPAIRS = NMAX // 2


@functools.partial(
    pl.kernel,
    out_type=(jax.ShapeDtypeStruct((NW * LANES,), jnp.float32),
              jax.ShapeDtypeStruct((NW * LANES,), jnp.float32)),
    mesh=plsc.VectorSubcoreMesh(core_axis_name="c", subcore_axis_name="s",
                                num_cores=NC, num_subcores=NS),
    scratch_types=(
        pltpu.VMEM((2, COLS, R), jnp.float32),
        pltpu.VMEM((2, COLS, R), jnp.float32),
        pltpu.VMEM((2, R), jnp.float32),
        pltpu.VMEM((N_TAIL * COLS,), jnp.float32),
        pltpu.VMEM((N_TAIL * COLS,), jnp.float32),
        pltpu.VMEM((N_TAIL,), jnp.float32),
        pltpu.VMEM((LANES,), jnp.float32),
        pltpu.VMEM((2 * LANES,), jnp.float32),
        pltpu.SemaphoreType.DMA,
        pltpu.SemaphoreType.DMA,
        pltpu.SemaphoreType.DMA,
    ),
    compiler_params=pltpu.CompilerParams(needs_layout_passes=False),
)
def _sc_loss(pred_hbm, target_hbm, iou_hbm, ptail_hbm, ttail_hbm,
             ws_out, cnt_out,
             pbuf, tbuf, ibuf, ptb, ttb, itb, mbuf, obuf, sem0, sem1, sem2):
  _sc_body(pred_hbm, target_hbm, iou_hbm, ptail_hbm, ttail_hbm,
           ws_out, cnt_out,
           pbuf, tbuf, ibuf, ptb, ttb, itb, mbuf, obuf, sem0, sem1, sem2)


def kernel(pred, target, iou):
  ptail = pred[N_MAIN:].reshape(-1)
  ttail = target[N_MAIN:].reshape(-1)
  ws, cnt = _sc_loss(pred.T, target.T, iou, ptail, ttail)
  return jnp.sum(ws) / jnp.sum(cnt)
